# Initial kernel scaffold; baseline (speedup 1.0000x reference)
import jax
import jax.numpy as jnp
from jax.experimental import pallas as pl


def kernel(sig):
    raise NotImplementedError("write your pallas kernel here")


# speedup vs baseline: 4.0210x; 4.0210x over previous
import functools

import jax
import jax.numpy as jnp
from jax import lax
from jax.experimental import pallas as pl
from jax.experimental.pallas import tpu as pltpu
from jax.experimental.pallas import tpu_sc as plsc

B = 64
N_SAMPLES = 64000
F = 512
STRIDE = 256
N_FRAMES = (N_SAMPLES - F) // STRIDE + 1
N_CHUNKS = N_SAMPLES // STRIDE

NUM_CORES = 2
NUM_SUBCORES = 16
NUM_WORKERS = NUM_CORES * NUM_SUBCORES
ROWS_PER_WORKER = B // NUM_WORKERS

_mesh = plsc.VectorSubcoreMesh(core_axis_name="c", subcore_axis_name="s")


@functools.partial(
    pl.kernel,
    mesh=_mesh,
    out_type=jax.ShapeDtypeStruct((B, 1, N_FRAMES, F), jnp.float32),
    scratch_types=[pltpu.VMEM((N_CHUNKS, STRIDE), jnp.float32)],
    compiler_params=pltpu.CompilerParams(use_tc_tiling_on_sc=False),
)
def _frames_kernel(sig_hbm, out_hbm, row_v):
    wid = lax.axis_index("s") * NUM_CORES + lax.axis_index("c")
    for r in range(ROWS_PER_WORKER):
        b = wid * ROWS_PER_WORKER + r
        pltpu.sync_copy(sig_hbm.at[b], row_v)
        pltpu.sync_copy(
            row_v.at[pl.ds(0, N_FRAMES)],
            out_hbm.at[b, 0, :, pl.ds(0, STRIDE)],
        )
        pltpu.sync_copy(
            row_v.at[pl.ds(1, N_FRAMES)],
            out_hbm.at[b, 0, :, pl.ds(STRIDE, STRIDE)],
        )


def kernel(sig):
    return _frames_kernel(sig.reshape(B, N_CHUNKS, STRIDE))


# speedup vs baseline: 4.0958x; 1.0186x over previous
import functools

import jax
import jax.numpy as jnp
from jax import lax
from jax.experimental import pallas as pl
from jax.experimental.pallas import tpu as pltpu
from jax.experimental.pallas import tpu_sc as plsc

B = 64
N_SAMPLES = 64000
F = 512
STRIDE = 256
N_FRAMES = (N_SAMPLES - F) // STRIDE + 1
N_CHUNKS = N_SAMPLES // STRIDE

NUM_CORES = 2
NUM_SUBCORES = 16
NUM_WORKERS = NUM_CORES * NUM_SUBCORES
ROWS_PER_WORKER = B // NUM_WORKERS

_mesh = plsc.VectorSubcoreMesh(core_axis_name="c", subcore_axis_name="s")


@functools.partial(
    pl.kernel,
    mesh=_mesh,
    out_type=jax.ShapeDtypeStruct((B, 1, N_FRAMES, F), jnp.float32),
    scratch_types=[
        pltpu.VMEM((N_CHUNKS, STRIDE), jnp.float32),
        pltpu.VMEM((N_CHUNKS, STRIDE), jnp.float32),
        pltpu.SemaphoreType.DMA,
        pltpu.SemaphoreType.DMA,
        pltpu.SemaphoreType.DMA,
    ],
    compiler_params=pltpu.CompilerParams(use_tc_tiling_on_sc=False),
)
def _frames_kernel(sig_hbm, out_hbm, buf0, buf1, sem_in, sem_o0, sem_o1):
    wid = lax.axis_index("s") * NUM_CORES + lax.axis_index("c")
    bufs = (buf0, buf1)
    out_sems = (sem_o0, sem_o1)
    stores = []
    for r in range(ROWS_PER_WORKER):
        b = wid * ROWS_PER_WORKER + r
        buf = bufs[r % 2]
        pltpu.async_copy(sig_hbm.at[b], buf, sem_in).wait()
        stores.append(
            pltpu.async_copy(
                buf.at[pl.ds(0, N_FRAMES)],
                out_hbm.at[b, 0, :, pl.ds(0, STRIDE)],
                out_sems[r % 2],
            )
        )
        stores.append(
            pltpu.async_copy(
                buf.at[pl.ds(1, N_FRAMES)],
                out_hbm.at[b, 0, :, pl.ds(STRIDE, STRIDE)],
                out_sems[r % 2],
            )
        )
    for cp in stores:
        cp.wait()


def kernel(sig):
    return _frames_kernel(sig.reshape(B, N_CHUNKS, STRIDE))
